# Initial kernel scaffold; baseline (speedup 1.0000x reference)
import jax
import jax.numpy as jnp
from jax.experimental import pallas as pl


def kernel(hidden_states, router_w, expert_gate_up, expert_out_w, shared_gate_w, shared_inter_w, shared_out_w, shared_expert_gate_w):
    raise NotImplementedError("write your pallas kernel here")


# speedup vs baseline: 1.0898x; 1.0898x over previous
import jax
import jax.numpy as jnp
from jax.experimental import pallas as pl
from jax.experimental.pallas import tpu as pltpu

HIDDEN = 2048
INTER = 512
INTER_SHARED = 2048
NUM_EXPERTS = 64
TOP_K = 8
NEG_INF = -1e30

J_SHARED = 4
CHUNK_SHARED = INTER_SHARED // J_SHARED


def _shared_router_kernel(x_ref, gw_ref, iw_ref, ow_ref, rw_ref, sgw_ref,
                          shared_out_ref, routing_ref):
    j = pl.program_id(0)
    x = x_ref[:]
    g = jax.nn.silu(jnp.dot(x, gw_ref[:], preferred_element_type=jnp.float32))
    i = jnp.dot(x, iw_ref[:], preferred_element_type=jnp.float32)
    contrib = jnp.dot(g * i, ow_ref[:], preferred_element_type=jnp.float32)

    @pl.when(j == 0)
    def _init():
        shared_out_ref[:] = contrib
        logits = jnp.dot(x, rw_ref[:], preferred_element_type=jnp.float32)
        iota = jax.lax.broadcasted_iota(jnp.int32, logits.shape, 1)
        vals = logits
        sel = jnp.zeros(logits.shape, jnp.bool_)
        for _ in range(TOP_K):
            m = jnp.max(vals, axis=-1, keepdims=True)
            cand = jnp.where(vals == m, iota, NUM_EXPERTS)
            idx = jnp.min(cand, axis=-1, keepdims=True)
            pick = iota == idx
            sel = jnp.logical_or(sel, pick)
            vals = jnp.where(pick, NEG_INF, vals)
        mtop = jnp.max(jnp.where(sel, logits, NEG_INF), axis=-1, keepdims=True)
        ex = jnp.where(sel, jnp.exp(logits - mtop), 0.0)
        routing_ref[:] = ex / jnp.sum(ex, axis=-1, keepdims=True)

    @pl.when(j > 0)
    def _acc():
        shared_out_ref[:] += contrib

    @pl.when(j == pl.num_programs(0) - 1)
    def _fin():
        sg = jax.nn.sigmoid(
            jnp.dot(x, sgw_ref[:], preferred_element_type=jnp.float32))
        shared_out_ref[:] *= sg


def _expert_kernel(x_ref, routing_ref, shared_ref, gu_ref, ow_ref, out_ref):
    e = pl.program_id(0)
    x = x_ref[:]
    xw = jnp.dot(x, gu_ref[0], preferred_element_type=jnp.float32)
    gate = xw[:, :INTER]
    up = xw[:, INTER:]
    h = up * jax.nn.silu(gate)
    iota = jax.lax.broadcasted_iota(jnp.int32, routing_ref.shape, 1)
    w = jnp.sum(jnp.where(iota == e, routing_ref[:], 0.0), axis=-1,
                keepdims=True)
    contrib = jnp.dot(h * w, ow_ref[0], preferred_element_type=jnp.float32)

    @pl.when(e == 0)
    def _init():
        out_ref[:] = shared_ref[:] + contrib

    @pl.when(e > 0)
    def _acc():
        out_ref[:] += contrib


def _moe(x, router_w, expert_gate_up, expert_out_w, shared_gate_w,
         shared_inter_w, shared_out_w, shared_expert_gate_w, interpret=False):
    T = x.shape[0]
    shared_part, routing = pl.pallas_call(
        _shared_router_kernel,
        grid=(J_SHARED,),
        in_specs=[
            pl.BlockSpec((T, HIDDEN), lambda j: (0, 0)),
            pl.BlockSpec((HIDDEN, CHUNK_SHARED), lambda j: (0, j)),
            pl.BlockSpec((HIDDEN, CHUNK_SHARED), lambda j: (0, j)),
            pl.BlockSpec((CHUNK_SHARED, HIDDEN), lambda j: (j, 0)),
            pl.BlockSpec((HIDDEN, NUM_EXPERTS), lambda j: (0, 0)),
            pl.BlockSpec((HIDDEN, 1), lambda j: (0, 0)),
        ],
        out_specs=[
            pl.BlockSpec((T, HIDDEN), lambda j: (0, 0)),
            pl.BlockSpec((T, NUM_EXPERTS), lambda j: (0, 0)),
        ],
        out_shape=[
            jax.ShapeDtypeStruct((T, HIDDEN), jnp.float32),
            jax.ShapeDtypeStruct((T, NUM_EXPERTS), jnp.float32),
        ],
        compiler_params=pltpu.CompilerParams(
            dimension_semantics=("arbitrary",)),
        interpret=interpret,
    )(x, shared_gate_w, shared_inter_w, shared_out_w, router_w,
      shared_expert_gate_w)

    out = pl.pallas_call(
        _expert_kernel,
        grid=(NUM_EXPERTS,),
        in_specs=[
            pl.BlockSpec((T, HIDDEN), lambda e: (0, 0)),
            pl.BlockSpec((T, NUM_EXPERTS), lambda e: (0, 0)),
            pl.BlockSpec((T, HIDDEN), lambda e: (0, 0)),
            pl.BlockSpec((1, HIDDEN, 2 * INTER), lambda e: (e, 0, 0)),
            pl.BlockSpec((1, INTER, HIDDEN), lambda e: (e, 0, 0)),
        ],
        out_specs=pl.BlockSpec((T, HIDDEN), lambda e: (0, 0)),
        out_shape=jax.ShapeDtypeStruct((T, HIDDEN), jnp.float32),
        compiler_params=pltpu.CompilerParams(
            dimension_semantics=("arbitrary",)),
        interpret=interpret,
    )(x, routing, shared_part, expert_gate_up, expert_out_w)
    return out


def kernel(hidden_states, router_w, expert_gate_up, expert_out_w,
           shared_gate_w, shared_inter_w, shared_out_w, shared_expert_gate_w):
    b, s, h = hidden_states.shape
    x = hidden_states.reshape(-1, h)
    out = _moe(x, router_w, expert_gate_up, expert_out_w, shared_gate_w,
               shared_inter_w, shared_out_w, shared_expert_gate_w)
    return out.reshape(b, s, h)
